# Initial kernel scaffold; baseline (speedup 1.0000x reference)
import jax
import jax.numpy as jnp
from jax.experimental import pallas as pl


def kernel(x, adj, W1, W_mu, W_sig, noise):
    raise NotImplementedError("write your pallas kernel here")


# speedup vs baseline: 1.2435x; 1.2435x over previous
import jax
import jax.numpy as jnp
from jax.experimental import pallas as pl


def _xw_kernel(x_ref, w1_ref, out_ref):
    out_ref[...] = jnp.dot(x_ref[...], w1_ref[...],
                           preferred_element_type=jnp.float32)


def _hw_kernel(adj_ref, xw_ref, wcat_ref, out_ref):
    h = jax.nn.relu(jnp.dot(adj_ref[...], xw_ref[...],
                            preferred_element_type=jnp.float32))
    out_ref[...] = jnp.dot(h, wcat_ref[...],
                           preferred_element_type=jnp.float32)


def _z_kernel(adj_ref, hw_ref, noise_ref, out_ref, *, zdim):
    t = jnp.dot(adj_ref[...], hw_ref[...], preferred_element_type=jnp.float32)
    mu = t[:, :zdim]
    log_sig = t[:, zdim:]
    out_ref[...] = mu + noise_ref[...] * jnp.exp(log_sig)


def _decode_kernel(zb_ref, z_ref, out_ref):
    out_ref[...] = jax.lax.dot_general(
        zb_ref[...], z_ref[...], (((1,), (1,)), ((), ())),
        preferred_element_type=jnp.float32)


def kernel(x, adj, W1, W_mu, W_sig, noise):
    n, d = x.shape
    h_dim = W1.shape[1]
    z_dim = W_mu.shape[1]
    bm = 400 if n % 400 == 0 else n

    wcat = jnp.concatenate([W_mu, W_sig], axis=1)

    xw = pl.pallas_call(
        _xw_kernel,
        out_shape=jax.ShapeDtypeStruct((n, h_dim), jnp.float32),
    )(x, W1)

    grid = (n // bm,)

    hw = pl.pallas_call(
        _hw_kernel,
        grid=grid,
        in_specs=[
            pl.BlockSpec((bm, n), lambda i: (i, 0)),
            pl.BlockSpec((n, h_dim), lambda i: (0, 0)),
            pl.BlockSpec((h_dim, 2 * z_dim), lambda i: (0, 0)),
        ],
        out_specs=pl.BlockSpec((bm, 2 * z_dim), lambda i: (i, 0)),
        out_shape=jax.ShapeDtypeStruct((n, 2 * z_dim), jnp.float32),
    )(adj, xw, wcat)

    import functools
    z = pl.pallas_call(
        functools.partial(_z_kernel, zdim=z_dim),
        grid=grid,
        in_specs=[
            pl.BlockSpec((bm, n), lambda i: (i, 0)),
            pl.BlockSpec((n, 2 * z_dim), lambda i: (0, 0)),
            pl.BlockSpec((bm, z_dim), lambda i: (i, 0)),
        ],
        out_specs=pl.BlockSpec((bm, z_dim), lambda i: (i, 0)),
        out_shape=jax.ShapeDtypeStruct((n, z_dim), jnp.float32),
    )(adj, hw, noise)

    out = pl.pallas_call(
        _decode_kernel,
        grid=grid,
        in_specs=[
            pl.BlockSpec((bm, z_dim), lambda i: (i, 0)),
            pl.BlockSpec((n, z_dim), lambda i: (0, 0)),
        ],
        out_specs=pl.BlockSpec((bm, n), lambda i: (i, 0)),
        out_shape=jax.ShapeDtypeStruct((n, n), jnp.float32),
    )(z, z)

    return out
